# baseline (speedup 1.0000x reference)
import functools

import jax
import jax.numpy as jnp
from jax import lax
from jax.experimental import pallas as pl
from jax.experimental.pallas import tpu as pltpu
from jax.experimental.pallas import tpu_sc as plsc

NC = 2
NS = 16
LANES = 16
NG = 128
EPS = 1e-5


def _softplus_sc(x):
  t = jnp.exp(-jnp.abs(x))
  z = t / (2.0 + t)
  z2 = z * z
  log1p = (2.0 * z) * (1.0 + z2 * (1.0 / 3.0 + z2 * (0.2 + z2 * (1.0 / 7.0))))
  return jnp.maximum(x, 0.0) + log1p


def _make_edge_kernel(N, E, H):
  NW = NC * NS
  CH = 40
  EW = E // NW
  n_chunks = EW // CH
  NP = ((N + NS * 64 - 1) // (NS * 64)) * (NS * 64)
  RPT = NP // NS
  ZR = CH
  n_zcopies = RPT // ZR
  G = H // LANES

  assert EW * NW == E and n_chunks * CH == EW
  assert RPT * NS == NP and n_zcopies * ZR == RPT

  mesh = plsc.VectorSubcoreMesh(core_axis_name="c", subcore_axis_name="s")

  out_type = [jax.ShapeDtypeStruct((NC, NP, H), jnp.float32)]

  scratch = [
      pltpu.VMEM((CH,), jnp.int32),
      pltpu.VMEM((CH,), jnp.int32),
      pltpu.VMEM((CH, 2 * H), jnp.float32),
      pltpu.VMEM((CH, 2 * H), jnp.float32),
      pltpu.VMEM((CH, 2 * H), jnp.float32),
      pltpu.VMEM((CH, H), jnp.float32),
      pltpu.VMEM_SHARED((NP, H), jnp.float32),
      pltpu.SemaphoreType.DMA,
      pltpu.SemaphoreType.DMA,
  ]

  def body(pd_hbm, ps_hbm, ef_hbm, dst_hbm, src_hbm, *rest):
    (agg_out, dsti, srci, dbuf, sbuf, ebuf, mbuf, agg_sh,
     sem1, sem2) = rest

    c = lax.axis_index("c")
    s = lax.axis_index("s")
    w = c * NS + s

    def zrow(i, _):
      for g in range(G):
        mbuf[i, pl.ds(g * LANES, LANES)] = jnp.zeros((LANES,), jnp.float32)
      return 0

    lax.fori_loop(0, ZR, zrow, 0)
    for k in range(n_zcopies):
      pltpu.sync_copy(mbuf, agg_sh.at[pl.ds(s * RPT + k * ZR, ZR), :])

    plsc.subcore_barrier()

    def chunk_body(g, _):
      base = w * EW + g * CH
      pltpu.sync_copy(dst_hbm.at[pl.ds(base, CH)], dsti)
      pltpu.sync_copy(src_hbm.at[pl.ds(base, CH)], srci)
      cp1 = pltpu.async_copy(pd_hbm.at[dsti], dbuf, sem1)
      cp2 = pltpu.async_copy(ps_hbm.at[srci], sbuf, sem2)
      pltpu.sync_copy(ef_hbm.at[pl.ds(base, CH), :], ebuf)
      cp1.wait()
      cp2.wait()

      def edge_body(i, _):
        for gg in range(G):
          lo = gg * LANES
          gf = (dbuf[i, pl.ds(lo, LANES)] + sbuf[i, pl.ds(lo, LANES)]
                + ebuf[i, pl.ds(lo, LANES)])
          gs = (dbuf[i, pl.ds(H + lo, LANES)] + sbuf[i, pl.ds(H + lo, LANES)]
                + ebuf[i, pl.ds(H + lo, LANES)])
          sig = 1.0 / (1.0 + jnp.exp(-gf))
          mbuf[i, pl.ds(lo, LANES)] = sig * _softplus_sc(gs)
        return 0

      lax.fori_loop(0, CH, edge_body, 0)

      pltpu.sync_copy(mbuf, agg_sh.at[dsti], add=True)
      return 0

    lax.fori_loop(0, n_chunks, chunk_body, 0)

    plsc.subcore_barrier()

    pltpu.sync_copy(agg_sh.at[pl.ds(s * RPT, RPT), :],
                    agg_out.at[c, pl.ds(s * RPT, RPT), :])

  return pl.kernel(body, out_type=out_type, mesh=mesh, scratch_types=scratch)


def _make_deg_kernel(N, E, H):
  NW = NC * NS
  CH = 80
  EW = E // NW
  n_chunks = EW // CH
  NP = ((N + NS * 64 - 1) // (NS * 64)) * (NS * 64)
  RPT = NP // NS
  ZR = CH
  n_zcopies = RPT // ZR
  G = H // LANES

  mesh = plsc.VectorSubcoreMesh(core_axis_name="c", subcore_axis_name="s")

  out_type = [jax.ShapeDtypeStruct((NC, NP, H), jnp.float32)]
  scratch = [
      pltpu.VMEM((CH,), jnp.int32),
      pltpu.VMEM((CH, H), jnp.float32),
      pltpu.VMEM_SHARED((NP, H), jnp.float32),
  ]

  def body(dst_hbm, *rest):
    (deg_out, dsti, ones, deg_sh) = rest

    c = lax.axis_index("c")
    s = lax.axis_index("s")
    w = c * NS + s

    def zrow(i, _):
      for g in range(G):
        ones[i, pl.ds(g * LANES, LANES)] = jnp.zeros((LANES,), jnp.float32)
      return 0
    lax.fori_loop(0, ZR, zrow, 0)
    for k in range(n_zcopies):
      pltpu.sync_copy(ones, deg_sh.at[pl.ds(s * RPT + k * ZR, ZR), :])

    def onerow(i, _):
      for g in range(G):
        ones[i, pl.ds(g * LANES, LANES)] = jnp.ones((LANES,), jnp.float32)
      return 0
    lax.fori_loop(0, CH, onerow, 0)

    plsc.subcore_barrier()

    def chunk_body(g, _):
      base = w * EW + g * CH
      pltpu.sync_copy(dst_hbm.at[pl.ds(base, CH)], dsti)
      pltpu.sync_copy(ones, deg_sh.at[dsti], add=True)
      return 0

    lax.fori_loop(0, n_chunks, chunk_body, 0)

    plsc.subcore_barrier()

    pltpu.sync_copy(deg_sh.at[pl.ds(s * RPT, RPT), :],
                    deg_out.at[c, pl.ds(s * RPT, RPT), :])

  return pl.kernel(body, out_type=out_type, mesh=mesh, scratch_types=scratch)


def _embed_body(x_ref, w_ref, b_ref, o_ref):
  o_ref[...] = (jnp.dot(x_ref[...], w_ref[...],
                        preferred_element_type=jnp.float32) + b_ref[...])


def _efes_body(ea_ref, w_ref, b_ref, o0, o1, o2, o3):
  z = jnp.dot(ea_ref[...], w_ref[...],
              preferred_element_type=jnp.float32) + b_ref[...]
  n = o0.shape[1]
  o0[...] = z[:, 0 * n:1 * n]
  o1[...] = z[:, 1 * n:2 * n]
  o2[...] = z[:, 2 * n:3 * n]
  o3[...] = z[:, 3 * n:4 * n]


def _proj_body(h_ref, wd_ref, ws_ref, pd_ref, ps_ref):
  h = h_ref[...]
  pd_ref[...] = jnp.dot(h, wd_ref[...], preferred_element_type=jnp.float32)
  ps_ref[...] = jnp.dot(h, ws_ref[...], preferred_element_type=jnp.float32)


def _post_body(h_ref, agg_ref, deg_ref, gamma_ref, beta_ref, o_ref):
  h = h_ref[...]
  agg = agg_ref[0] + agg_ref[1]
  deg = deg_ref[0, :, :1] + deg_ref[1, :, :1]
  dinv = 1.0 / jnp.maximum(deg, 1.0)
  hc = h + agg * dinv
  n = hc.shape[0]
  mean = jnp.sum(hc, axis=0, keepdims=True) / n
  d = hc - mean
  var = jnp.sum(d * d, axis=0, keepdims=True) / n
  hn = d * jax.lax.rsqrt(var + EPS) * gamma_ref[...] + beta_ref[...]
  o_ref[...] = jnp.maximum(hn, 0.0) + h


def _pool_body(h_ref, batch_ref, w1_ref, b1_ref, w2_ref, b2_ref, o_ref):
  h = h_ref[...]
  ng = o_ref.shape[0]
  n = h.shape[0]
  iota = lax.broadcasted_iota(jnp.int32, (ng, n), 0)
  onehot = jnp.where(iota == batch_ref[...], 1.0, 0.0)
  counts = jnp.sum(onehot, axis=1, keepdims=True)
  sums = jnp.dot(onehot, h, preferred_element_type=jnp.float32)
  pooled = sums / jnp.maximum(counts, 1.0)
  o = jnp.dot(pooled, w1_ref[...], preferred_element_type=jnp.float32)
  o = jax.nn.softplus(o + b1_ref[...])
  o = jnp.dot(o, w2_ref[...], preferred_element_type=jnp.float32)
  o_ref[...] = o + b2_ref[...]


def kernel(x, edge_index, edge_attr, batch, params):
  N, D = x.shape
  E = edge_index.shape[1]
  H = params['emb_W'].shape[1]
  ED = edge_attr.shape[1]
  NCONV = 4

  src = edge_index[0].astype(jnp.int32)
  dst = edge_index[1].astype(jnp.int32)
  batchi = batch.astype(jnp.int32).reshape(1, N)

  wd, wsr, we, be = [], [], [], []
  for l in range(NCONV):
    Wf = params[f'conv{l}_Wf']
    Ws = params[f'conv{l}_Ws']
    wd.append(jnp.concatenate([Wf[:H], Ws[:H]], axis=1))
    wsr.append(jnp.concatenate([Wf[H:2 * H], Ws[H:2 * H]], axis=1))
    we.append(jnp.concatenate([Wf[2 * H:], Ws[2 * H:]], axis=1))
    be.append(jnp.concatenate([params[f'conv{l}_bf'], params[f'conv{l}_bs']]))
  we_all = jnp.concatenate(we, axis=1)
  be_all = jnp.concatenate(be).reshape(1, 8 * H)

  f32 = jnp.float32

  h = pl.pallas_call(
      _embed_body,
      out_shape=jax.ShapeDtypeStruct((N, H), f32),
  )(x, params['emb_W'], params['emb_b'].reshape(1, H))

  BE = 2000
  efes = pl.pallas_call(
      _efes_body,
      grid=(E // BE,),
      in_specs=[
          pl.BlockSpec((BE, ED), lambda i: (i, 0)),
          pl.BlockSpec((ED, 8 * H), lambda i: (0, 0)),
          pl.BlockSpec((1, 8 * H), lambda i: (0, 0)),
      ],
      out_specs=[pl.BlockSpec((BE, 2 * H), lambda i: (i, 0))] * NCONV,
      out_shape=[jax.ShapeDtypeStruct((E, 2 * H), f32)] * NCONV,
  )(edge_attr, we_all, be_all)

  edge_call = _make_edge_kernel(N, E, H)
  deg_call = _make_deg_kernel(N, E, H)
  (degp,) = deg_call(dst)
  degp = degp[:, :N]

  NB = 5
  for l in range(NCONV):
    pd, ps = pl.pallas_call(
        _proj_body,
        grid=(NB,),
        in_specs=[
            pl.BlockSpec((N // NB, H), lambda i: (i, 0)),
            pl.BlockSpec((H, 2 * H), lambda i: (0, 0)),
            pl.BlockSpec((H, 2 * H), lambda i: (0, 0)),
        ],
        out_specs=[pl.BlockSpec((N // NB, 2 * H), lambda i: (i, 0))] * 2,
        out_shape=[jax.ShapeDtypeStruct((N, 2 * H), f32)] * 2,
    )(h, wd[l], wsr[l])

    (aggp,) = edge_call(pd, ps, efes[l], dst, src)
    aggp = aggp[:, :N]

    h = pl.pallas_call(
        _post_body,
        out_shape=jax.ShapeDtypeStruct((N, H), f32),
    )(h, aggp, degp, params[f'conv{l}_gamma'].reshape(1, H),
      params[f'conv{l}_beta'].reshape(1, H))

  out = pl.pallas_call(
      _pool_body,
      out_shape=jax.ShapeDtypeStruct((NG, 1), f32),
  )(h, batchi, params['head_W1'], params['head_b1'].reshape(1, H // 2),
    params['head_W2'], params['head_b2'].reshape(1, 1))

  return out[:, 0]


# speedup vs baseline: 2.7137x; 2.7137x over previous
import functools

import jax
import jax.numpy as jnp
from jax import lax
from jax.experimental import pallas as pl
from jax.experimental.pallas import tpu as pltpu
from jax.experimental.pallas import tpu_sc as plsc

NC = 2
NS = 16
LANES = 16
NG = 128
EPS = 1e-5


def _softplus_sc(x):
  t = jnp.exp(-jnp.abs(x))
  z = t / (2.0 + t)
  z2 = z * z
  log1p = (2.0 * z) * (1.0 + z2 * (1.0 / 3.0 + z2 * (0.2 + z2 * (1.0 / 7.0))))
  return jnp.maximum(x, 0.0) + log1p


def _make_edge_kernel(N, E, H):
  NW = NC * NS
  CH = 40
  EW = E // NW
  n_pairs = EW // (2 * CH)
  NP = ((N + NS * 64 - 1) // (NS * 64)) * (NS * 64)
  RPT = NP // NS
  ZR = CH
  n_zcopies = RPT // ZR
  G = H // LANES

  assert EW * NW == E and n_pairs * 2 * CH == EW
  assert RPT * NS == NP and n_zcopies * ZR == RPT

  mesh = plsc.VectorSubcoreMesh(core_axis_name="c", subcore_axis_name="s")

  out_type = [jax.ShapeDtypeStruct((NC, NP, H), jnp.float32)]

  scratch = [
      pltpu.VMEM((CH,), jnp.int32),
      pltpu.VMEM((CH,), jnp.int32),
      pltpu.VMEM((CH,), jnp.int32),
      pltpu.VMEM((CH, 2 * H), jnp.float32),
      pltpu.VMEM((CH, 2 * H), jnp.float32),
      pltpu.VMEM((CH, 2 * H), jnp.float32),
      pltpu.VMEM((CH, H), jnp.float32),
      pltpu.VMEM((CH, H), jnp.float32),
      pltpu.VMEM_SHARED((NP, H), jnp.float32),
      pltpu.SemaphoreType.DMA,
      pltpu.SemaphoreType.DMA,
      pltpu.SemaphoreType.DMA,
      pltpu.SemaphoreType.DMA,
  ]

  def body(pd_hbm, ps_hbm, ef_hbm, dst_hbm, src_hbm, *rest):
    (agg_out, dsti0, dsti1, srci, dbuf, sbuf, ebuf, mbuf0, mbuf1, agg_sh,
     sem1, sem2, sem3, sem4) = rest

    c = lax.axis_index("c")
    s = lax.axis_index("s")
    w = c * NS + s

    @plsc.parallel_loop(0, ZR)
    def _(i):
      for g in range(G):
        mbuf0[i, pl.ds(g * LANES, LANES)] = jnp.zeros((LANES,), jnp.float32)

    for k in range(n_zcopies):
      pltpu.sync_copy(mbuf0, agg_sh.at[pl.ds(s * RPT + k * ZR, ZR), :])

    plsc.subcore_barrier()

    def do_chunk(base, dsti, mbuf):
      pltpu.sync_copy(dst_hbm.at[pl.ds(base, CH)], dsti)
      pltpu.sync_copy(src_hbm.at[pl.ds(base, CH)], srci)
      cp1 = pltpu.async_copy(pd_hbm.at[dsti], dbuf, sem1)
      cp2 = pltpu.async_copy(ps_hbm.at[srci], sbuf, sem2)
      pltpu.sync_copy(ef_hbm.at[pl.ds(base, CH), :], ebuf)
      cp1.wait()
      cp2.wait()

      @plsc.parallel_loop(0, CH, unroll=2)
      def _(i):
        for gg in range(G):
          lo = gg * LANES
          gf = (dbuf[i, pl.ds(lo, LANES)] + sbuf[i, pl.ds(lo, LANES)]
                + ebuf[i, pl.ds(lo, LANES)])
          gs = (dbuf[i, pl.ds(H + lo, LANES)] + sbuf[i, pl.ds(H + lo, LANES)]
                + ebuf[i, pl.ds(H + lo, LANES)])
          sig = 1.0 / (1.0 + jnp.exp(-gf))
          mbuf[i, pl.ds(lo, LANES)] = sig * _softplus_sc(gs)

    def pair_body(p, _):
      base = w * EW + p * (2 * CH)
      do_chunk(base, dsti0, mbuf0)
      scat0 = pltpu.async_copy(mbuf0, agg_sh.at[dsti0], sem3, add=True)
      do_chunk(base + CH, dsti1, mbuf1)
      scat1 = pltpu.async_copy(mbuf1, agg_sh.at[dsti1], sem4, add=True)
      scat0.wait()
      scat1.wait()
      return 0

    lax.fori_loop(0, n_pairs, pair_body, 0)

    plsc.subcore_barrier()

    pltpu.sync_copy(agg_sh.at[pl.ds(s * RPT, RPT), :],
                    agg_out.at[c, pl.ds(s * RPT, RPT), :])

  return pl.kernel(body, out_type=out_type, mesh=mesh, scratch_types=scratch)


def _make_deg_kernel(N, E, H):
  NW = NC * NS
  CH = 80
  EW = E // NW
  n_chunks = EW // CH
  NP = ((N + NS * 64 - 1) // (NS * 64)) * (NS * 64)
  RPT = NP // NS
  ZR = CH
  n_zcopies = RPT // ZR
  G = H // LANES

  mesh = plsc.VectorSubcoreMesh(core_axis_name="c", subcore_axis_name="s")

  out_type = [jax.ShapeDtypeStruct((NC, NP, H), jnp.float32)]
  scratch = [
      pltpu.VMEM((CH,), jnp.int32),
      pltpu.VMEM((CH, H), jnp.float32),
      pltpu.VMEM_SHARED((NP, H), jnp.float32),
  ]

  def body(dst_hbm, *rest):
    (deg_out, dsti, ones, deg_sh) = rest

    c = lax.axis_index("c")
    s = lax.axis_index("s")
    w = c * NS + s

    def zrow(i, _):
      for g in range(G):
        ones[i, pl.ds(g * LANES, LANES)] = jnp.zeros((LANES,), jnp.float32)
      return 0
    lax.fori_loop(0, ZR, zrow, 0)
    for k in range(n_zcopies):
      pltpu.sync_copy(ones, deg_sh.at[pl.ds(s * RPT + k * ZR, ZR), :])

    def onerow(i, _):
      for g in range(G):
        ones[i, pl.ds(g * LANES, LANES)] = jnp.ones((LANES,), jnp.float32)
      return 0
    lax.fori_loop(0, CH, onerow, 0)

    plsc.subcore_barrier()

    def chunk_body(g, _):
      base = w * EW + g * CH
      pltpu.sync_copy(dst_hbm.at[pl.ds(base, CH)], dsti)
      pltpu.sync_copy(ones, deg_sh.at[dsti], add=True)
      return 0

    lax.fori_loop(0, n_chunks, chunk_body, 0)

    plsc.subcore_barrier()

    pltpu.sync_copy(deg_sh.at[pl.ds(s * RPT, RPT), :],
                    deg_out.at[c, pl.ds(s * RPT, RPT), :])

  return pl.kernel(body, out_type=out_type, mesh=mesh, scratch_types=scratch)


def _embed_body(x_ref, w_ref, b_ref, o_ref):
  o_ref[...] = (jnp.dot(x_ref[...], w_ref[...],
                        preferred_element_type=jnp.float32) + b_ref[...])


def _efes_body(ea_ref, w_ref, b_ref, o0, o1, o2, o3):
  z = jnp.dot(ea_ref[...], w_ref[...],
              preferred_element_type=jnp.float32) + b_ref[...]
  n = o0.shape[1]
  o0[...] = z[:, 0 * n:1 * n]
  o1[...] = z[:, 1 * n:2 * n]
  o2[...] = z[:, 2 * n:3 * n]
  o3[...] = z[:, 3 * n:4 * n]


def _proj_body(h_ref, wd_ref, ws_ref, pd_ref, ps_ref):
  h = h_ref[...]
  pd_ref[...] = jnp.dot(h, wd_ref[...], preferred_element_type=jnp.float32)
  ps_ref[...] = jnp.dot(h, ws_ref[...], preferred_element_type=jnp.float32)


def _post_body(h_ref, agg_ref, deg_ref, gamma_ref, beta_ref, o_ref):
  h = h_ref[...]
  agg = agg_ref[0] + agg_ref[1]
  deg = deg_ref[0, :, :1] + deg_ref[1, :, :1]
  dinv = 1.0 / jnp.maximum(deg, 1.0)
  hc = h + agg * dinv
  n = hc.shape[0]
  mean = jnp.sum(hc, axis=0, keepdims=True) / n
  d = hc - mean
  var = jnp.sum(d * d, axis=0, keepdims=True) / n
  hn = d * jax.lax.rsqrt(var + EPS) * gamma_ref[...] + beta_ref[...]
  o_ref[...] = jnp.maximum(hn, 0.0) + h


def _pool_body(h_ref, batch_ref, w1_ref, b1_ref, w2_ref, b2_ref, o_ref):
  h = h_ref[...]
  ng = o_ref.shape[0]
  n = h.shape[0]
  iota = lax.broadcasted_iota(jnp.int32, (ng, n), 0)
  onehot = jnp.where(iota == batch_ref[...], 1.0, 0.0)
  counts = jnp.sum(onehot, axis=1, keepdims=True)
  sums = jnp.dot(onehot, h, preferred_element_type=jnp.float32)
  pooled = sums / jnp.maximum(counts, 1.0)
  o = jnp.dot(pooled, w1_ref[...], preferred_element_type=jnp.float32)
  o = jax.nn.softplus(o + b1_ref[...])
  o = jnp.dot(o, w2_ref[...], preferred_element_type=jnp.float32)
  o_ref[...] = o + b2_ref[...]


def kernel(x, edge_index, edge_attr, batch, params):
  N, D = x.shape
  E = edge_index.shape[1]
  H = params['emb_W'].shape[1]
  ED = edge_attr.shape[1]
  NCONV = 4

  src = edge_index[0].astype(jnp.int32)
  dst = edge_index[1].astype(jnp.int32)
  batchi = batch.astype(jnp.int32).reshape(1, N)

  wd, wsr, we, be = [], [], [], []
  for l in range(NCONV):
    Wf = params[f'conv{l}_Wf']
    Ws = params[f'conv{l}_Ws']
    wd.append(jnp.concatenate([Wf[:H], Ws[:H]], axis=1))
    wsr.append(jnp.concatenate([Wf[H:2 * H], Ws[H:2 * H]], axis=1))
    we.append(jnp.concatenate([Wf[2 * H:], Ws[2 * H:]], axis=1))
    be.append(jnp.concatenate([params[f'conv{l}_bf'], params[f'conv{l}_bs']]))
  we_all = jnp.concatenate(we, axis=1)
  be_all = jnp.concatenate(be).reshape(1, 8 * H)

  f32 = jnp.float32

  h = pl.pallas_call(
      _embed_body,
      out_shape=jax.ShapeDtypeStruct((N, H), f32),
  )(x, params['emb_W'], params['emb_b'].reshape(1, H))

  BE = 2000
  efes = pl.pallas_call(
      _efes_body,
      grid=(E // BE,),
      in_specs=[
          pl.BlockSpec((BE, ED), lambda i: (i, 0)),
          pl.BlockSpec((ED, 8 * H), lambda i: (0, 0)),
          pl.BlockSpec((1, 8 * H), lambda i: (0, 0)),
      ],
      out_specs=[pl.BlockSpec((BE, 2 * H), lambda i: (i, 0))] * NCONV,
      out_shape=[jax.ShapeDtypeStruct((E, 2 * H), f32)] * NCONV,
  )(edge_attr, we_all, be_all)

  edge_call = _make_edge_kernel(N, E, H)
  deg_call = _make_deg_kernel(N, E, H)
  (degp,) = deg_call(dst)
  degp = degp[:, :N]

  NB = 5
  for l in range(NCONV):
    pd, ps = pl.pallas_call(
        _proj_body,
        grid=(NB,),
        in_specs=[
            pl.BlockSpec((N // NB, H), lambda i: (i, 0)),
            pl.BlockSpec((H, 2 * H), lambda i: (0, 0)),
            pl.BlockSpec((H, 2 * H), lambda i: (0, 0)),
        ],
        out_specs=[pl.BlockSpec((N // NB, 2 * H), lambda i: (i, 0))] * 2,
        out_shape=[jax.ShapeDtypeStruct((N, 2 * H), f32)] * 2,
    )(h, wd[l], wsr[l])

    (aggp,) = edge_call(pd, ps, efes[l], dst, src)
    aggp = aggp[:, :N]

    h = pl.pallas_call(
        _post_body,
        out_shape=jax.ShapeDtypeStruct((N, H), f32),
    )(h, aggp, degp, params[f'conv{l}_gamma'].reshape(1, H),
      params[f'conv{l}_beta'].reshape(1, H))

  out = pl.pallas_call(
      _pool_body,
      out_shape=jax.ShapeDtypeStruct((NG, 1), f32),
  )(h, batchi, params['head_W1'], params['head_b1'].reshape(1, H // 2),
    params['head_W2'], params['head_b2'].reshape(1, 1))

  return out[:, 0]
